# baseline (speedup 1.0000x reference)
import functools

import jax
import jax.numpy as jnp
from jax import lax
from jax.experimental import pallas as pl
from jax.experimental.pallas import tpu as pltpu
from jax.experimental.pallas import tpu_sc as plsc

H = 1024
NV = H // 16
CH = 16
EPS = 1e-12


def _tc_tables_kernel(pos_ref, a_ref, b_ref,
                      sq_pos_ref, sq_a_ref, sq_b_ref,
                      c_pa_ref, c_pb_ref, c_ab_ref):
    pos = pos_ref[...]
    at = a_ref[...]
    bt = b_ref[...]
    sq_pos_ref[0, :] = jnp.sum(pos, axis=1)
    sq_pos_ref[1, :] = jnp.sum(pos * pos, axis=1)
    sq_a_ref[0, :] = jnp.sum(at, axis=1)
    sq_a_ref[1, :] = jnp.sum(at * at, axis=1)
    sq_b_ref[0, :] = jnp.sum(bt, axis=1)
    sq_b_ref[1, :] = jnp.sum(bt * bt, axis=1)
    dims = (((1,), (1,)), ((), ()))
    c_pa_ref[...] = lax.dot_general(pos, at, dims,
                                    preferred_element_type=jnp.float32)
    c_pb_ref[...] = lax.dot_general(pos, bt, dims,
                                    preferred_element_type=jnp.float32)
    c_ab_ref[...] = lax.dot_general(at, bt, dims,
                                    preferred_element_type=jnp.float32)


def _tc_stats_kernel(pa_ref, sp_ref, sq_pos_ref, sq_a_ref, sq_b_ref,
                     c_pa_ref, c_pb_ref, c_ab_ref, mu_ref, rs_ref):
    na = sq_a_ref.shape[1]
    pa_f = pa_ref[0, 0, :]
    sp_f = sp_ref[0, 0, :]
    iota_a = lax.broadcasted_iota(jnp.int32, (1, na), 1)
    oha = (pa_f[:, None] == iota_a).astype(jnp.float32)
    ohb = (sp_f[:, None] == iota_a).astype(jnp.float32)
    cdot = (((1,), (0,)), ((), ()))
    dims = (((1,), (1,)), ((), ()))
    s_ab = (lax.dot_general(oha, sq_a_ref[0, :], cdot,
                            preferred_element_type=jnp.float32)
            + lax.dot_general(ohb, sq_b_ref[0, :], cdot,
                              preferred_element_type=jnp.float32))
    q_ab = (lax.dot_general(oha, sq_a_ref[1, :], cdot,
                            preferred_element_type=jnp.float32)
            + lax.dot_general(ohb, sq_b_ref[1, :], cdot,
                              preferred_element_type=jnp.float32))
    t_pa = jnp.sum(oha * c_pa_ref[...], axis=1)
    t_pb = jnp.sum(ohb * c_pb_ref[...], axis=1)
    d = lax.dot_general(ohb, c_ab_ref[...], dims,
                        preferred_element_type=jnp.float32)
    t_ab = jnp.sum(oha * d, axis=1)
    mu = (sq_pos_ref[0, :] + s_ab) * (1.0 / H)
    ex2 = (sq_pos_ref[1, :] + q_ab + 2.0 * (t_pa + t_pb + t_ab)) * (1.0 / H)
    var = ex2 - mu * mu
    mu_ref[0, 0, :] = mu
    rs_ref[0, 0, :] = lax.rsqrt(var + EPS)


def _tc_stats(pos_table, a_table, b_table, pa, sp):
    n_batch, n_sent = pa.shape
    n_pos = pos_table.shape[0]
    na = a_table.shape[0]
    sq_pos, sq_a, sq_b, c_pa, c_pb, c_ab = pl.pallas_call(
        _tc_tables_kernel,
        out_shape=(
            jax.ShapeDtypeStruct((2, n_pos), jnp.float32),
            jax.ShapeDtypeStruct((2, na), jnp.float32),
            jax.ShapeDtypeStruct((2, na), jnp.float32),
            jax.ShapeDtypeStruct((n_pos, na), jnp.float32),
            jax.ShapeDtypeStruct((n_pos, na), jnp.float32),
            jax.ShapeDtypeStruct((na, na), jnp.float32),
        ),
    )(pos_table, a_table, b_table)
    full2 = lambda shape: pl.BlockSpec(shape, lambda i: (0, 0))
    mu3, rs3 = pl.pallas_call(
        _tc_stats_kernel,
        grid=(n_batch,),
        in_specs=[
            pl.BlockSpec((1, 1, n_sent), lambda i: (i, 0, 0)),
            pl.BlockSpec((1, 1, n_sent), lambda i: (i, 0, 0)),
            full2(sq_pos.shape),
            full2(sq_a.shape),
            full2(sq_b.shape),
            full2(c_pa.shape),
            full2(c_pb.shape),
            full2(c_ab.shape),
        ],
        out_specs=(
            pl.BlockSpec((1, 1, n_sent), lambda i: (i, 0, 0)),
            pl.BlockSpec((1, 1, n_sent), lambda i: (i, 0, 0)),
        ),
        out_shape=(
            jax.ShapeDtypeStruct((n_batch, 1, n_sent), jnp.float32),
            jax.ShapeDtypeStruct((n_batch, 1, n_sent), jnp.float32),
        ),
    )(pa.reshape(n_batch, 1, n_sent), sp.reshape(n_batch, 1, n_sent),
      sq_pos, sq_a, sq_b, c_pa, c_pb, c_ab)
    return mu3.reshape(n_batch, n_sent), rs3.reshape(n_batch, n_sent)


def _make_sc_call(n_rows, rows_per_worker):
    n_chunks = rows_per_worker // CH
    chunks_per_batch = 512 // CH
    n_groups = n_chunks // 2
    assert n_groups == chunks_per_batch
    mesh = plsc.VectorSubcoreMesh(core_axis_name="c", subcore_axis_name="s")

    @functools.partial(
        pl.kernel,
        mesh=mesh,
        out_type=jax.ShapeDtypeStruct((n_rows, H), jnp.float32),
        scratch_types=[
            pltpu.VMEM((rows_per_worker,), jnp.int32),
            pltpu.VMEM((rows_per_worker,), jnp.int32),
            pltpu.VMEM((rows_per_worker,), jnp.float32),
            pltpu.VMEM((rows_per_worker,), jnp.float32),
            pltpu.VMEM((CH, H), jnp.float32),
            pltpu.VMEM((CH, H), jnp.float32),
            pltpu.VMEM((CH, H), jnp.float32),
            pltpu.VMEM((CH, H), jnp.float32),
            pltpu.VMEM((CH, H), jnp.float32),
            pltpu.VMEM((CH, H), jnp.float32),
            pltpu.VMEM((CH, H), jnp.float32),
            pltpu.VMEM((H,), jnp.float32),
            pltpu.VMEM((H,), jnp.float32),
            pltpu.SemaphoreType.DMA,
            pltpu.SemaphoreType.DMA,
            pltpu.SemaphoreType.DMA,
            pltpu.SemaphoreType.DMA,
            pltpu.SemaphoreType.DMA,
            pltpu.SemaphoreType.DMA,
        ],
    )
    def sc_call(pa_hbm, sp_hbm, pos_hbm, a_hbm, b_hbm, w_hbm, bias_hbm,
                mu_hbm, rs_hbm, out_hbm,
                pa_v, sp_v, mu_v, rs_v, a0, a1, b0, b1, posb, o0, o1,
                w_v, bias_v, sa0, sa1, sb0, sb1, so0, so1):
        wid = lax.axis_index("s") * 2 + lax.axis_index("c")
        base = wid * rows_per_worker
        pltpu.sync_copy(pa_hbm.at[pl.ds(base, rows_per_worker)], pa_v)
        pltpu.sync_copy(sp_hbm.at[pl.ds(base, rows_per_worker)], sp_v)
        pltpu.sync_copy(mu_hbm.at[pl.ds(base, rows_per_worker)], mu_v)
        pltpu.sync_copy(rs_hbm.at[pl.ds(base, rows_per_worker)], rs_v)
        pltpu.sync_copy(w_hbm, w_v)
        pltpu.sync_copy(bias_hbm, bias_v)

        def start_gather(c, ab, bb, sa, sb):
            ipa = pa_v[pl.ds(c * CH, CH)]
            isp = sp_v[pl.ds(c * CH, CH)]
            pltpu.async_copy(a_hbm.at[ipa], ab, sa)
            pltpu.async_copy(b_hbm.at[isp], bb, sb)

        def wait_gather(ab, bb, sa, sb):
            ipa = pa_v[pl.ds(0, CH)]
            pltpu.make_async_copy(a_hbm.at[ipa], ab, sa).wait()
            pltpu.make_async_copy(b_hbm.at[ipa], bb, sb).wait()

        def compute_chunk(c, ar, br, ob):
            muv = mu_v[pl.ds(c * CH, CH)]
            rsv = rs_v[pl.ds(c * CH, CH)]
            mus = [muv[r] for r in range(CH)]
            rss = [rsv[r] for r in range(CH)]

            @plsc.parallel_loop(0, H, 16, unroll=3)
            def col_body(o):
                sl = pl.ds(o, 16)
                wv = w_v[sl]
                bv = bias_v[sl]
                for r in range(CH):
                    x = ar[r, sl] + br[r, sl] + posb[r, sl]
                    ob[r, sl] = (x - mus[r]) * rss[r] * wv + bv

        def wait_out(ob, so):
            pltpu.make_async_copy(ob, out_hbm.at[pl.ds(base, CH)], so).wait()

        start_gather(0, a0, b0, sa0, sb0)

        def group_body(g, carry):
            c0 = g
            c1 = g + chunks_per_batch
            start_gather(c1, a1, b1, sa1, sb1)
            pltpu.sync_copy(pos_hbm.at[pl.ds(g * CH, CH)], posb)
            wait_gather(a0, b0, sa0, sb0)

            @pl.when(g > 0)
            def _():
                wait_out(o0, so0)

            compute_chunk(c0, a0, b0, o0)
            pltpu.async_copy(o0, out_hbm.at[pl.ds(base + c0 * CH, CH)], so0)

            @pl.when(g < n_groups - 1)
            def _():
                start_gather(g + 1, a0, b0, sa0, sb0)

            wait_gather(a1, b1, sa1, sb1)

            @pl.when(g > 0)
            def _():
                wait_out(o1, so1)

            compute_chunk(c1, a1, b1, o1)
            pltpu.async_copy(o1, out_hbm.at[pl.ds(base + c1 * CH, CH)], so1)
            return carry

        lax.fori_loop(0, n_groups, group_body, 0)
        wait_out(o0, so0)
        wait_out(o1, so1)

    return sc_call


def kernel(top_vecs, sent_struct_vec, pos_table, a_table, b_table, ln_w, ln_b):
    b, s, h = top_vecs.shape
    ssv = sent_struct_vec.astype(jnp.int32)
    pa = ssv[:, :, 0]
    sp = ssv[:, :, 1]
    mu, rstd = _tc_stats(pos_table, a_table, b_table, pa, sp)
    n_rows = b * s
    sc_call = _make_sc_call(n_rows, n_rows // 32)
    out = sc_call(pa.reshape(-1), sp.reshape(-1), pos_table, a_table,
                  b_table, ln_w, ln_b, mu.reshape(-1), rstd.reshape(-1))
    return out.reshape(b, s, h)


# speedup vs baseline: 1.0100x; 1.0100x over previous
import functools

import jax
import jax.numpy as jnp
from jax import lax
from jax.experimental import pallas as pl
from jax.experimental.pallas import tpu as pltpu
from jax.experimental.pallas import tpu_sc as plsc

H = 1024
NV = H // 16
CH = 16
EPS = 1e-12


def _tc_tables_kernel(pos_ref, a_ref, b_ref,
                      sq_pos_ref, sq_a_ref, sq_b_ref,
                      c_pa_ref, c_pb_ref, c_ab_ref):
    pos = pos_ref[...]
    at = a_ref[...]
    bt = b_ref[...]
    sq_pos_ref[0, :] = jnp.sum(pos, axis=1)
    sq_pos_ref[1, :] = jnp.sum(pos * pos, axis=1)
    sq_a_ref[0, :] = jnp.sum(at, axis=1)
    sq_a_ref[1, :] = jnp.sum(at * at, axis=1)
    sq_b_ref[0, :] = jnp.sum(bt, axis=1)
    sq_b_ref[1, :] = jnp.sum(bt * bt, axis=1)
    dims = (((1,), (1,)), ((), ()))
    c_pa_ref[...] = lax.dot_general(pos, at, dims,
                                    preferred_element_type=jnp.float32)
    c_pb_ref[...] = lax.dot_general(pos, bt, dims,
                                    preferred_element_type=jnp.float32)
    c_ab_ref[...] = lax.dot_general(at, bt, dims,
                                    preferred_element_type=jnp.float32)


def _tc_stats_kernel(pa_ref, sp_ref, sq_pos_ref, sq_a_ref, sq_b_ref,
                     c_pa_ref, c_pb_ref, c_ab_ref, mu_ref, rs_ref):
    na = sq_a_ref.shape[1]
    pa_f = pa_ref[0, 0, :]
    sp_f = sp_ref[0, 0, :]
    iota_a = lax.broadcasted_iota(jnp.int32, (1, na), 1)
    oha = (pa_f[:, None] == iota_a).astype(jnp.float32)
    ohb = (sp_f[:, None] == iota_a).astype(jnp.float32)
    cdot = (((1,), (0,)), ((), ()))
    dims = (((1,), (1,)), ((), ()))
    s_ab = (lax.dot_general(oha, sq_a_ref[0, :], cdot,
                            preferred_element_type=jnp.float32)
            + lax.dot_general(ohb, sq_b_ref[0, :], cdot,
                              preferred_element_type=jnp.float32))
    q_ab = (lax.dot_general(oha, sq_a_ref[1, :], cdot,
                            preferred_element_type=jnp.float32)
            + lax.dot_general(ohb, sq_b_ref[1, :], cdot,
                              preferred_element_type=jnp.float32))
    t_pa = jnp.sum(oha * c_pa_ref[...], axis=1)
    t_pb = jnp.sum(ohb * c_pb_ref[...], axis=1)
    d = lax.dot_general(ohb, c_ab_ref[...], dims,
                        preferred_element_type=jnp.float32)
    t_ab = jnp.sum(oha * d, axis=1)
    mu = (sq_pos_ref[0, :] + s_ab) * (1.0 / H)
    ex2 = (sq_pos_ref[1, :] + q_ab + 2.0 * (t_pa + t_pb + t_ab)) * (1.0 / H)
    var = ex2 - mu * mu
    mu_ref[0, 0, :] = mu
    rs_ref[0, 0, :] = lax.rsqrt(var + EPS)


def _tc_stats(pos_table, a_table, b_table, pa, sp):
    n_batch, n_sent = pa.shape
    n_pos = pos_table.shape[0]
    na = a_table.shape[0]
    sq_pos, sq_a, sq_b, c_pa, c_pb, c_ab = pl.pallas_call(
        _tc_tables_kernel,
        out_shape=(
            jax.ShapeDtypeStruct((2, n_pos), jnp.float32),
            jax.ShapeDtypeStruct((2, na), jnp.float32),
            jax.ShapeDtypeStruct((2, na), jnp.float32),
            jax.ShapeDtypeStruct((n_pos, na), jnp.float32),
            jax.ShapeDtypeStruct((n_pos, na), jnp.float32),
            jax.ShapeDtypeStruct((na, na), jnp.float32),
        ),
    )(pos_table, a_table, b_table)
    full2 = lambda shape: pl.BlockSpec(shape, lambda i: (0, 0))
    mu3, rs3 = pl.pallas_call(
        _tc_stats_kernel,
        grid=(n_batch,),
        in_specs=[
            pl.BlockSpec((1, 1, n_sent), lambda i: (i, 0, 0)),
            pl.BlockSpec((1, 1, n_sent), lambda i: (i, 0, 0)),
            full2(sq_pos.shape),
            full2(sq_a.shape),
            full2(sq_b.shape),
            full2(c_pa.shape),
            full2(c_pb.shape),
            full2(c_ab.shape),
        ],
        out_specs=(
            pl.BlockSpec((1, 1, n_sent), lambda i: (i, 0, 0)),
            pl.BlockSpec((1, 1, n_sent), lambda i: (i, 0, 0)),
        ),
        out_shape=(
            jax.ShapeDtypeStruct((n_batch, 1, n_sent), jnp.float32),
            jax.ShapeDtypeStruct((n_batch, 1, n_sent), jnp.float32),
        ),
    )(pa.reshape(n_batch, 1, n_sent), sp.reshape(n_batch, 1, n_sent),
      sq_pos, sq_a, sq_b, c_pa, c_pb, c_ab)
    return mu3.reshape(n_batch, n_sent), rs3.reshape(n_batch, n_sent)


def _make_sc_call(n_rows, rows_per_worker):
    n_chunks = rows_per_worker // CH
    chunks_per_batch = 512 // CH
    n_groups = n_chunks // 2
    assert n_groups == chunks_per_batch
    mesh = plsc.VectorSubcoreMesh(core_axis_name="c", subcore_axis_name="s")

    @functools.partial(
        pl.kernel,
        mesh=mesh,
        out_type=jax.ShapeDtypeStruct((n_rows, H), jnp.float32),
        scratch_types=[
            pltpu.VMEM((rows_per_worker,), jnp.int32),
            pltpu.VMEM((rows_per_worker,), jnp.int32),
            pltpu.VMEM((rows_per_worker,), jnp.float32),
            pltpu.VMEM((rows_per_worker,), jnp.float32),
            pltpu.VMEM((CH, H), jnp.float32),
            pltpu.VMEM((CH, H), jnp.float32),
            pltpu.VMEM((CH, H), jnp.float32),
            pltpu.VMEM((CH, H), jnp.float32),
            pltpu.VMEM((CH, H), jnp.float32),
            pltpu.VMEM((CH, H), jnp.float32),
            pltpu.VMEM((CH, H), jnp.float32),
            pltpu.VMEM((H,), jnp.float32),
            pltpu.VMEM((H,), jnp.float32),
            pltpu.SemaphoreType.DMA,
            pltpu.SemaphoreType.DMA,
            pltpu.SemaphoreType.DMA,
            pltpu.SemaphoreType.DMA,
            pltpu.SemaphoreType.DMA,
            pltpu.SemaphoreType.DMA,
            pltpu.SemaphoreType.DMA,
        ],
    )
    def sc_call(pa_hbm, sp_hbm, pos_hbm, a_hbm, b_hbm, w_hbm, bias_hbm,
                mu_hbm, rs_hbm, out_hbm,
                pa_v, sp_v, mu_v, rs_v, a0, a1, b0, b1, posb, o0, o1,
                w_v, bias_v, sa0, sa1, sb0, sb1, so0, so1, spos):
        wid = lax.axis_index("s") * 2 + lax.axis_index("c")
        base = wid * rows_per_worker
        pltpu.sync_copy(pa_hbm.at[pl.ds(base, rows_per_worker)], pa_v)
        pltpu.sync_copy(sp_hbm.at[pl.ds(base, rows_per_worker)], sp_v)
        pltpu.sync_copy(mu_hbm.at[pl.ds(base, rows_per_worker)], mu_v)
        pltpu.sync_copy(rs_hbm.at[pl.ds(base, rows_per_worker)], rs_v)
        pltpu.sync_copy(w_hbm, w_v)
        pltpu.sync_copy(bias_hbm, bias_v)

        def start_gather(c, ab, bb, sa, sb):
            ipa = pa_v[pl.ds(c * CH, CH)]
            isp = sp_v[pl.ds(c * CH, CH)]
            pltpu.async_copy(a_hbm.at[ipa], ab, sa)
            pltpu.async_copy(b_hbm.at[isp], bb, sb)

        def wait_gather(ab, bb, sa, sb):
            ipa = pa_v[pl.ds(0, CH)]
            pltpu.make_async_copy(a_hbm.at[ipa], ab, sa).wait()
            pltpu.make_async_copy(b_hbm.at[ipa], bb, sb).wait()

        def compute_chunk(c, ar, br, ob):
            muv = mu_v[pl.ds(c * CH, CH)]
            rsv = rs_v[pl.ds(c * CH, CH)]
            mus = [muv[r] for r in range(CH)]
            rss = [rsv[r] for r in range(CH)]

            @plsc.parallel_loop(0, H, 16, unroll=2)
            def col_body(o):
                sl = pl.ds(o, 16)
                wv = w_v[sl]
                bv = bias_v[sl]
                for r in range(CH):
                    x = ar[r, sl] + br[r, sl] + posb[r, sl]
                    ob[r, sl] = (x - mus[r]) * rss[r] * wv + bv

        def wait_out(ob, so):
            pltpu.make_async_copy(ob, out_hbm.at[pl.ds(base, CH)], so).wait()

        start_gather(0, a0, b0, sa0, sb0)

        def group_body(g, carry):
            c0 = g
            c1 = g + chunks_per_batch
            start_gather(c1, a1, b1, sa1, sb1)
            pltpu.async_copy(pos_hbm.at[pl.ds(g * CH, CH)], posb, spos)
            wait_gather(a0, b0, sa0, sb0)
            pltpu.make_async_copy(pos_hbm.at[pl.ds(0, CH)], posb, spos).wait()

            @pl.when(g > 0)
            def _():
                wait_out(o0, so0)

            compute_chunk(c0, a0, b0, o0)
            pltpu.async_copy(o0, out_hbm.at[pl.ds(base + c0 * CH, CH)], so0)

            @pl.when(g < n_groups - 1)
            def _():
                start_gather(g + 1, a0, b0, sa0, sb0)

            wait_gather(a1, b1, sa1, sb1)

            @pl.when(g > 0)
            def _():
                wait_out(o1, so1)

            compute_chunk(c1, a1, b1, o1)
            pltpu.async_copy(o1, out_hbm.at[pl.ds(base + c1 * CH, CH)], so1)
            return carry

        lax.fori_loop(0, n_groups, group_body, 0)
        wait_out(o0, so0)
        wait_out(o1, so1)

    return sc_call


def kernel(top_vecs, sent_struct_vec, pos_table, a_table, b_table, ln_w, ln_b):
    b, s, h = top_vecs.shape
    ssv = sent_struct_vec.astype(jnp.int32)
    pa = ssv[:, :, 0]
    sp = ssv[:, :, 1]
    mu, rstd = _tc_stats(pos_table, a_table, b_table, pa, sp)
    n_rows = b * s
    sc_call = _make_sc_call(n_rows, n_rows // 32)
    out = sc_call(pa.reshape(-1), sp.reshape(-1), pos_table, a_table,
                  b_table, ln_w, ln_b, mu.reshape(-1), rstd.reshape(-1))
    return out.reshape(b, s, h)


# speedup vs baseline: 1.0132x; 1.0032x over previous
import functools

import jax
import jax.numpy as jnp
from jax import lax
from jax.experimental import pallas as pl
from jax.experimental.pallas import tpu as pltpu
from jax.experimental.pallas import tpu_sc as plsc

H = 1024
NV = H // 16
CH = 16
EPS = 1e-12


def _tc_stats_kernel(pos_ref, a_ref, b_ref, pa_ref, sp_ref, mu_ref, rs_ref,
                     sq_pos_ref, sq_a_ref, sq_b_ref,
                     c_pa_ref, c_pb_ref, c_ab_ref):
    gb = pa_ref.shape[1]
    na = a_ref.shape[0]

    @pl.when(pl.program_id(0) == 0)
    def _():
        pos = pos_ref[...]
        at = a_ref[...]
        bt = b_ref[...]
        sq_pos_ref[0, :] = jnp.sum(pos, axis=1)
        sq_pos_ref[1, :] = jnp.sum(pos * pos, axis=1)
        sq_a_ref[0, :] = jnp.sum(at, axis=1)
        sq_a_ref[1, :] = jnp.sum(at * at, axis=1)
        sq_b_ref[0, :] = jnp.sum(bt, axis=1)
        sq_b_ref[1, :] = jnp.sum(bt * bt, axis=1)
        dims = (((1,), (1,)), ((), ()))
        c_pa_ref[...] = lax.dot_general(pos, at, dims,
                                        preferred_element_type=jnp.float32)
        c_pb_ref[...] = lax.dot_general(pos, bt, dims,
                                        preferred_element_type=jnp.float32)
        c_ab_ref[...] = lax.dot_general(at, bt, dims,
                                        preferred_element_type=jnp.float32)

    iota_a = lax.broadcasted_iota(jnp.int32, (1, na), 1)
    cdot = (((1,), (0,)), ((), ()))
    dims = (((1,), (1,)), ((), ()))

    def batch_body(j, carry):
        pa_f = pa_ref[0, j, :]
        sp_f = sp_ref[0, j, :]
        oha = (pa_f[:, None] == iota_a).astype(jnp.float32)
        ohb = (sp_f[:, None] == iota_a).astype(jnp.float32)
        s_ab = (lax.dot_general(oha, sq_a_ref[0, :], cdot,
                                preferred_element_type=jnp.float32)
                + lax.dot_general(ohb, sq_b_ref[0, :], cdot,
                                  preferred_element_type=jnp.float32))
        q_ab = (lax.dot_general(oha, sq_a_ref[1, :], cdot,
                                preferred_element_type=jnp.float32)
                + lax.dot_general(ohb, sq_b_ref[1, :], cdot,
                                  preferred_element_type=jnp.float32))
        t_pa = jnp.sum(oha * c_pa_ref[...], axis=1)
        t_pb = jnp.sum(ohb * c_pb_ref[...], axis=1)
        d = lax.dot_general(ohb, c_ab_ref[...], dims,
                            preferred_element_type=jnp.float32)
        t_ab = jnp.sum(oha * d, axis=1)
        mu = (sq_pos_ref[0, :] + s_ab) * (1.0 / H)
        ex2 = (sq_pos_ref[1, :] + q_ab
               + 2.0 * (t_pa + t_pb + t_ab)) * (1.0 / H)
        var = ex2 - mu * mu
        mu_ref[0, j, :] = mu
        rs_ref[0, j, :] = lax.rsqrt(var + EPS)
        return carry

    lax.fori_loop(0, gb, batch_body, 0)


def _tc_stats(pos_table, a_table, b_table, pa, sp):
    n_batch, n_sent = pa.shape
    n_pos = pos_table.shape[0]
    na = a_table.shape[0]
    gb = 8
    full2 = lambda shape: pl.BlockSpec(shape, lambda i: (0, 0))
    mu3, rs3 = pl.pallas_call(
        _tc_stats_kernel,
        grid=(n_batch // gb,),
        in_specs=[
            full2(pos_table.shape),
            full2(a_table.shape),
            full2(b_table.shape),
            pl.BlockSpec((1, gb, n_sent), lambda i: (i, 0, 0)),
            pl.BlockSpec((1, gb, n_sent), lambda i: (i, 0, 0)),
        ],
        out_specs=(
            pl.BlockSpec((1, gb, n_sent), lambda i: (i, 0, 0)),
            pl.BlockSpec((1, gb, n_sent), lambda i: (i, 0, 0)),
        ),
        out_shape=(
            jax.ShapeDtypeStruct((n_batch // gb, gb, n_sent), jnp.float32),
            jax.ShapeDtypeStruct((n_batch // gb, gb, n_sent), jnp.float32),
        ),
        scratch_shapes=[
            pltpu.VMEM((2, n_pos), jnp.float32),
            pltpu.VMEM((2, na), jnp.float32),
            pltpu.VMEM((2, na), jnp.float32),
            pltpu.VMEM((n_pos, na), jnp.float32),
            pltpu.VMEM((n_pos, na), jnp.float32),
            pltpu.VMEM((na, na), jnp.float32),
        ],
    )(pos_table, a_table, b_table,
      pa.reshape(n_batch // gb, gb, n_sent),
      sp.reshape(n_batch // gb, gb, n_sent))
    return mu3.reshape(n_batch, n_sent), rs3.reshape(n_batch, n_sent)


def _make_sc_call(n_rows, rows_per_worker):
    n_chunks = rows_per_worker // CH
    chunks_per_batch = 512 // CH
    n_groups = n_chunks // 2
    assert n_groups == chunks_per_batch
    mesh = plsc.VectorSubcoreMesh(core_axis_name="c", subcore_axis_name="s")

    @functools.partial(
        pl.kernel,
        mesh=mesh,
        out_type=jax.ShapeDtypeStruct((n_rows, H), jnp.float32),
        scratch_types=[
            pltpu.VMEM((rows_per_worker,), jnp.int32),
            pltpu.VMEM((rows_per_worker,), jnp.int32),
            pltpu.VMEM((rows_per_worker,), jnp.float32),
            pltpu.VMEM((rows_per_worker,), jnp.float32),
            pltpu.VMEM((CH, H), jnp.float32),
            pltpu.VMEM((CH, H), jnp.float32),
            pltpu.VMEM((CH, H), jnp.float32),
            pltpu.VMEM((CH, H), jnp.float32),
            pltpu.VMEM((CH, H), jnp.float32),
            pltpu.VMEM((CH, H), jnp.float32),
            pltpu.VMEM((CH, H), jnp.float32),
            pltpu.VMEM((H,), jnp.float32),
            pltpu.VMEM((H,), jnp.float32),
            pltpu.SemaphoreType.DMA,
            pltpu.SemaphoreType.DMA,
            pltpu.SemaphoreType.DMA,
            pltpu.SemaphoreType.DMA,
            pltpu.SemaphoreType.DMA,
            pltpu.SemaphoreType.DMA,
            pltpu.SemaphoreType.DMA,
        ],
    )
    def sc_call(pa_hbm, sp_hbm, pos_hbm, a_hbm, b_hbm, w_hbm, bias_hbm,
                mu_hbm, rs_hbm, out_hbm,
                pa_v, sp_v, mu_v, rs_v, a0, a1, b0, b1, posb, o0, o1,
                w_v, bias_v, sa0, sa1, sb0, sb1, so0, so1, spos):
        wid = lax.axis_index("s") * 2 + lax.axis_index("c")
        base = wid * rows_per_worker
        pltpu.sync_copy(pa_hbm.at[pl.ds(base, rows_per_worker)], pa_v)
        pltpu.sync_copy(sp_hbm.at[pl.ds(base, rows_per_worker)], sp_v)
        pltpu.sync_copy(mu_hbm.at[pl.ds(base, rows_per_worker)], mu_v)
        pltpu.sync_copy(rs_hbm.at[pl.ds(base, rows_per_worker)], rs_v)
        pltpu.sync_copy(w_hbm, w_v)
        pltpu.sync_copy(bias_hbm, bias_v)

        def start_gather(c, ab, bb, sa, sb):
            ipa = pa_v[pl.ds(c * CH, CH)]
            isp = sp_v[pl.ds(c * CH, CH)]
            pltpu.async_copy(a_hbm.at[ipa], ab, sa)
            pltpu.async_copy(b_hbm.at[isp], bb, sb)

        def wait_gather(ab, bb, sa, sb):
            ipa = pa_v[pl.ds(0, CH)]
            pltpu.make_async_copy(a_hbm.at[ipa], ab, sa).wait()
            pltpu.make_async_copy(b_hbm.at[ipa], bb, sb).wait()

        def compute_chunk(c, ar, br, ob):
            muv = mu_v[pl.ds(c * CH, CH)]
            rsv = rs_v[pl.ds(c * CH, CH)]
            mus = [muv[r] for r in range(CH)]
            rss = [rsv[r] for r in range(CH)]

            @plsc.parallel_loop(0, H, 16, unroll=2)
            def col_body(o):
                sl = pl.ds(o, 16)
                wv = w_v[sl]
                bv = bias_v[sl]
                for r in range(CH):
                    x = ar[r, sl] + br[r, sl] + posb[r, sl]
                    ob[r, sl] = (x - mus[r]) * rss[r] * wv + bv

        def wait_out(ob, so):
            pltpu.make_async_copy(ob, out_hbm.at[pl.ds(base, CH)], so).wait()

        start_gather(0, a0, b0, sa0, sb0)

        def group_body(g, carry):
            c0 = g
            c1 = g + chunks_per_batch
            start_gather(c1, a1, b1, sa1, sb1)
            pltpu.async_copy(pos_hbm.at[pl.ds(g * CH, CH)], posb, spos)
            wait_gather(a0, b0, sa0, sb0)
            pltpu.make_async_copy(pos_hbm.at[pl.ds(0, CH)], posb, spos).wait()

            @pl.when(g > 0)
            def _():
                wait_out(o0, so0)

            compute_chunk(c0, a0, b0, o0)
            pltpu.async_copy(o0, out_hbm.at[pl.ds(base + c0 * CH, CH)], so0)

            @pl.when(g < n_groups - 1)
            def _():
                start_gather(g + 1, a0, b0, sa0, sb0)

            wait_gather(a1, b1, sa1, sb1)

            @pl.when(g > 0)
            def _():
                wait_out(o1, so1)

            compute_chunk(c1, a1, b1, o1)
            pltpu.async_copy(o1, out_hbm.at[pl.ds(base + c1 * CH, CH)], so1)
            return carry

        lax.fori_loop(0, n_groups, group_body, 0)
        wait_out(o0, so0)
        wait_out(o1, so1)

    return sc_call


def kernel(top_vecs, sent_struct_vec, pos_table, a_table, b_table, ln_w, ln_b):
    b, s, h = top_vecs.shape
    ssv = sent_struct_vec.astype(jnp.int32)
    pa = ssv[:, :, 0]
    sp = ssv[:, :, 1]
    mu, rstd = _tc_stats(pos_table, a_table, b_table, pa, sp)
    n_rows = b * s
    sc_call = _make_sc_call(n_rows, n_rows // 32)
    out = sc_call(pa.reshape(-1), sp.reshape(-1), pos_table, a_table,
                  b_table, ln_w, ln_b, mu.reshape(-1), rstd.reshape(-1))
    return out.reshape(b, s, h)
